# Initial kernel scaffold; baseline (speedup 1.0000x reference)
import jax
import jax.numpy as jnp
from jax.experimental import pallas as pl


def kernel(inputs, Wr, br, W1, b1, W2, b2):
    raise NotImplementedError("write your pallas kernel here")


# speedup vs baseline: 2.4205x; 2.4205x over previous
import functools
import math

import jax
import jax.numpy as jnp
from jax.experimental import pallas as pl
from jax.experimental.pallas import tpu as pltpu


def _moe_kernel(x32_ref, xbf_ref, wr_ref, br_ref, w1_ref, b1_ref, w2_ref,
                b2_ref, out_ref, router_ref, *, tile_t, tile_h, n_e):
    e = pl.program_id(0)
    h = pl.program_id(1)
    t = pl.program_id(2)

    @pl.when((e == 0) & (h == 0) & (t == 0))
    def _prologue():
        logits = jnp.dot(x32_ref[...], wr_ref[...],
                         preferred_element_type=jnp.float32,
                         precision=jax.lax.Precision.DEFAULT) + br_ref[...]
        m = jnp.max(logits, axis=1, keepdims=True)
        ex = jnp.exp(logits - m)
        p = ex / jnp.sum(ex, axis=1, keepdims=True)
        iota = jax.lax.broadcasted_iota(jnp.int32, p.shape, 1)
        m1 = jnp.max(p, axis=1, keepdims=True)
        idx1 = jnp.min(jnp.where(p == m1, iota, n_e), axis=1, keepdims=True)
        sel1 = iota == idx1
        pm = jnp.where(sel1, -jnp.inf, p)
        m2 = jnp.max(pm, axis=1, keepdims=True)
        idx2 = jnp.min(jnp.where(pm == m2, iota, n_e), axis=1, keepdims=True)
        sel = sel1 | (iota == idx2)
        ew = jnp.where(sel, jnp.exp(p - m1), 0.0)
        router_ref[...] = ew / jnp.sum(ew, axis=1, keepdims=True)
        out_ref[...] = jnp.zeros_like(out_ref)

    xs = xbf_ref[pl.ds(t * tile_t, tile_t), :]
    mid = jnp.dot(xs, w1_ref[0], preferred_element_type=jnp.float32)
    mid = mid + b1_ref[0]
    mid = 0.5 * mid * (1.0 + jax.lax.erf(mid * (1.0 / math.sqrt(2.0))))
    y = jnp.dot(mid.astype(jnp.bfloat16), w2_ref[0],
                preferred_element_type=jnp.float32)
    y = y + jnp.where(h == 0, b2_ref[0], jnp.zeros_like(b2_ref[0]))

    rt = router_ref[pl.ds(t * tile_t, tile_t), :]
    lane = jax.lax.broadcasted_iota(jnp.int32, rt.shape, 1)
    w = jnp.sum(jnp.where(lane == e, rt, 0.0), axis=1, keepdims=True)
    out_ref[pl.ds(t * tile_t, tile_t), :] += w * y


def kernel(inputs, Wr, br, W1, b1, W2, b2):
    B, T, D = inputs.shape
    E = Wr.shape[1]
    H = W1.shape[2]
    x32 = inputs.reshape(T, D)
    xbf = x32.astype(jnp.bfloat16)
    w1b = W1.astype(jnp.bfloat16)
    w2b = W2.astype(jnp.bfloat16)
    br2 = br.reshape(1, E)
    b1r = b1.reshape(E, 1, H)
    b2r = b2.reshape(E, 1, D)

    tile_t = min(512, T)
    tile_h = min(512, H)
    n_t = T // tile_t
    n_h = H // tile_h

    out = pl.pallas_call(
        functools.partial(_moe_kernel, tile_t=tile_t, tile_h=tile_h, n_e=E),
        grid=(E, n_h, n_t),
        in_specs=[
            pl.BlockSpec((T, D), lambda e, h, t: (0, 0)),
            pl.BlockSpec((T, D), lambda e, h, t: (0, 0)),
            pl.BlockSpec((D, E), lambda e, h, t: (0, 0)),
            pl.BlockSpec((1, E), lambda e, h, t: (0, 0)),
            pl.BlockSpec((1, D, tile_h), lambda e, h, t: (e, 0, h)),
            pl.BlockSpec((1, 1, tile_h), lambda e, h, t: (e, 0, h)),
            pl.BlockSpec((1, tile_h, D), lambda e, h, t: (e, h, 0)),
            pl.BlockSpec((1, 1, D), lambda e, h, t: (e, 0, 0)),
        ],
        out_specs=pl.BlockSpec((T, D), lambda e, h, t: (0, 0)),
        out_shape=jax.ShapeDtypeStruct((T, D), jnp.float32),
        scratch_shapes=[pltpu.VMEM((T, E), jnp.float32)],
        compiler_params=pltpu.CompilerParams(
            dimension_semantics=("arbitrary", "arbitrary", "arbitrary"),
            vmem_limit_bytes=100 * 1024 * 1024,
        ),
    )(x32, xbf, Wr, br2, w1b, b1r, w2b, b2r)
    return out.reshape(B, T, D)
